# baseline (speedup 1.0000x reference)
import functools

import jax
import jax.numpy as jnp
from jax.experimental import pallas as pl
from jax.experimental.pallas import tpu as pltpu

_B, _D, _N = 4, 3, 4096
_SN = 256


def _nn_loss_body(src_ref, tgt_ref, out_ref):
    b = pl.program_id(0)
    i = pl.program_id(1)
    s = src_ref[0]
    t = tgt_ref[0]
    sb = s.astype(jnp.bfloat16)
    tb = t.astype(jnp.bfloat16)
    cross = jax.lax.dot_general(
        sb, tb, (((0,), (0,)), ((), ())),
        preferred_element_type=jnp.float32,
    )
    s2 = jnp.sum(s * s, axis=0)
    t2 = jnp.sum(t * t, axis=0)
    d2_sel = s2[:, None] + t2[None, :] - 2.0 * cross
    row_min_sel = jnp.min(d2_sel, axis=1)
    dx = s[0, :, None] - t[0, None, :]
    dy = s[1, :, None] - t[1, None, :]
    dz = s[2, :, None] - t[2, None, :]
    d2 = dx * dx + dy * dy + dz * dz
    chosen = jnp.where(d2_sel == row_min_sel[:, None], d2, jnp.inf)
    row_val = jnp.min(chosen, axis=1)
    partial = jnp.sum(row_val) * (1.0 / (3.0 * _N))

    @pl.when((b == 0) & (i == 0))
    def _():
        out_ref[...] = jnp.zeros_like(out_ref)

    row = jax.lax.broadcasted_iota(jnp.int32, (_B, 128), 0)
    col = jax.lax.broadcasted_iota(jnp.int32, (_B, 128), 1)
    mask = (row == b) & (col == 0)
    out_ref[...] += jnp.where(mask, partial, 0.0)


def kernel(source_point_cloud, target_point_cloud):
    grid = (_B, _N // _SN)
    out = pl.pallas_call(
        _nn_loss_body,
        grid=grid,
        in_specs=[
            pl.BlockSpec((1, _D, _SN), lambda b, i: (b, 0, i)),
            pl.BlockSpec((1, _D, _N), lambda b, i: (b, 0, 0)),
        ],
        out_specs=pl.BlockSpec((_B, 128), lambda b, i: (0, 0)),
        out_shape=jax.ShapeDtypeStruct((_B, 128), jnp.float32),
    )(source_point_cloud, target_point_cloud)
    return out[:, 0]


# speedup vs baseline: 1.0795x; 1.0795x over previous
import functools

import jax
import jax.numpy as jnp
from jax.experimental import pallas as pl
from jax.experimental.pallas import tpu as pltpu

_B, _D, _N = 4, 3, 4096
_SN = 256


def _nn_loss_body(src_ref, tgt_ref, out_ref):
    b = pl.program_id(0)
    i = pl.program_id(1)
    s = src_ref[0]
    t = tgt_ref[0]
    sb = s.astype(jnp.bfloat16)
    tb = t.astype(jnp.bfloat16)
    cross = jax.lax.dot_general(
        sb, tb, (((0,), (0,)), ((), ())),
        preferred_element_type=jnp.float32,
    )
    s2 = jnp.sum(s * s, axis=0)
    t2 = jnp.sum(t * t, axis=0)
    base = s2[:, None] + t2[None, :]
    d2_sel = base - 2.0 * cross
    row_min_sel = jnp.min(d2_sel, axis=1)
    cross_f32 = (
        s[0, :, None] * t[0, None, :]
        + s[1, :, None] * t[1, None, :]
        + s[2, :, None] * t[2, None, :]
    )
    d2 = base - 2.0 * cross_f32
    chosen = jnp.where(d2_sel == row_min_sel[:, None], d2, jnp.inf)
    row_val = jnp.min(chosen, axis=1)
    partial = jnp.sum(row_val) * (1.0 / (3.0 * _N))

    @pl.when((b == 0) & (i == 0))
    def _():
        out_ref[...] = jnp.zeros_like(out_ref)

    row = jax.lax.broadcasted_iota(jnp.int32, (_B, 128), 0)
    col = jax.lax.broadcasted_iota(jnp.int32, (_B, 128), 1)
    mask = (row == b) & (col == 0)
    out_ref[...] += jnp.where(mask, partial, 0.0)


def kernel(source_point_cloud, target_point_cloud):
    grid = (_B, _N // _SN)
    out = pl.pallas_call(
        _nn_loss_body,
        grid=grid,
        in_specs=[
            pl.BlockSpec((1, _D, _SN), lambda b, i: (b, 0, i)),
            pl.BlockSpec((1, _D, _N), lambda b, i: (b, 0, 0)),
        ],
        out_specs=pl.BlockSpec((_B, 128), lambda b, i: (0, 0)),
        out_shape=jax.ShapeDtypeStruct((_B, 128), jnp.float32),
    )(source_point_cloud, target_point_cloud)
    return out[:, 0]


# speedup vs baseline: 1.4911x; 1.3814x over previous
import functools

import jax
import jax.numpy as jnp
from jax import lax
from jax.experimental import pallas as pl
from jax.experimental.pallas import tpu as pltpu
from jax.experimental.pallas import tpu_sc as plsc

_B, _D, _N = 4, 3, 4096
_SN = 512

_NC, _NS, _L = 2, 16, 16
_NW = _NC * _NS
_CPW = _B * _N // _NW


def _select_body(src_ref, tgt_ref, idx_ref):
    s = src_ref[0]
    t = tgt_ref[0]
    sb = s.astype(jnp.bfloat16)
    tb = t.astype(jnp.bfloat16)
    cross = jax.lax.dot_general(
        sb, tb, (((0,), (0,)), ((), ())),
        preferred_element_type=jnp.float32,
    )
    s2 = jnp.sum(s * s, axis=0)
    t2 = jnp.sum(t * t, axis=0)
    d2_sel = (s2[:, None] + t2[None, :]) - 2.0 * cross
    row_min = jnp.min(d2_sel, axis=1)
    col = jax.lax.broadcasted_iota(jnp.int32, (_SN, _N), 1)
    first = jnp.min(
        jnp.where(d2_sel == row_min[:, None], col, _N), axis=1
    )
    idx_ref[0, 0, :] = first


def _select_indices(source_point_cloud, target_point_cloud):
    grid = (_B, _N // _SN)
    idx = pl.pallas_call(
        _select_body,
        grid=grid,
        in_specs=[
            pl.BlockSpec((1, _D, _SN), lambda b, i: (b, 0, i)),
            pl.BlockSpec((1, _D, _N), lambda b, i: (b, 0, 0)),
        ],
        out_specs=pl.BlockSpec((1, 1, _SN), lambda b, i: (b, 0, i)),
        out_shape=jax.ShapeDtypeStruct((_B, 1, _N), jnp.int32),
    )(source_point_cloud, target_point_cloud)
    return idx.reshape(_B, _N)


_SC_MESH = plsc.VectorSubcoreMesh(
    core_axis_name="c", subcore_axis_name="s", num_cores=_NC, num_subcores=_NS
)


@functools.partial(
    pl.kernel,
    mesh=_SC_MESH,
    compiler_params=pltpu.CompilerParams(needs_layout_passes=False),
    out_type=jax.ShapeDtypeStruct((_NW * _L,), jnp.float32),
    scratch_types=[
        [pltpu.VMEM((_N,), jnp.float32) for _ in range(_D)],
        [pltpu.VMEM((_CPW,), jnp.float32) for _ in range(_D)],
        pltpu.VMEM((_CPW,), jnp.int32),
        pltpu.VMEM((_L,), jnp.float32),
    ],
)
def _sc_rescore(src_hbm, tgt_hbm, idx_hbm, out_hbm, tgt_v, src_v, idx_v, acc_v):
    wid = lax.axis_index("s") * _NC + lax.axis_index("c")
    chunks_per_batch = _N // _CPW
    b = wid // chunks_per_batch
    chunk = wid % chunks_per_batch
    base = chunk * _CPW
    for c in range(_D):
        pltpu.sync_copy(tgt_hbm.at[pl.ds((b * _D + c) * _N, _N)], tgt_v[c])
        pltpu.sync_copy(
            src_hbm.at[pl.ds((b * _D + c) * _N + base, _CPW)], src_v[c]
        )
    pltpu.sync_copy(idx_hbm.at[pl.ds(b * _N + base, _CPW)], idx_v)

    def body(j, acc):
        iv = idx_v[pl.ds(j * _L, _L)]
        for c in range(_D):
            tc = plsc.load_gather(tgt_v[c], [iv])
            sc = src_v[c][pl.ds(j * _L, _L)]
            d = sc - tc
            acc = acc + d * d
        return acc

    acc = lax.fori_loop(0, _CPW // _L, body, jnp.zeros((_L,), jnp.float32))
    acc_v[...] = acc
    pltpu.sync_copy(acc_v, out_hbm.at[pl.ds(wid * _L, _L)])


def kernel(source_point_cloud, target_point_cloud):
    idx = _select_indices(source_point_cloud, target_point_cloud)
    partials = _sc_rescore(
        source_point_cloud.reshape(-1),
        target_point_cloud.reshape(-1),
        idx.reshape(-1),
    )
    loss = partials.reshape(_B, (_N // _CPW) * _L).sum(axis=1) * (
        1.0 / (3.0 * _N)
    )
    return loss


# speedup vs baseline: 1.7369x; 1.1649x over previous
import functools

import jax
import jax.numpy as jnp
from jax import lax
from jax.experimental import pallas as pl
from jax.experimental.pallas import tpu as pltpu
from jax.experimental.pallas import tpu_sc as plsc

_B, _D, _N = 4, 3, 4096
_SN = 512

_NC, _NS, _L = 2, 16, 16
_NW = _NC * _NS
_CPW = _B * _N // _NW


def _select_body(src_ref, tgt_ref, colf_ref, idx_ref):
    s = src_ref[0]
    t = tgt_ref[0]
    sb = s.astype(jnp.bfloat16)
    tb = t.astype(jnp.bfloat16)
    cross = jax.lax.dot_general(
        sb, tb, (((0,), (0,)), ((), ())),
        preferred_element_type=jnp.float32,
    )
    s2 = jnp.sum(s * s, axis=0)
    t2 = jnp.sum(t * t, axis=0)
    d2_sel = (-2.0) * cross + (s2[:, None] + t2[None, :])
    row_min = jnp.min(d2_sel, axis=1)
    colf = colf_ref[0]
    first = jnp.min(
        jnp.where(d2_sel == row_min[:, None], colf[None, :], float(_N)), axis=1
    )
    idx_ref[0, 0, :] = first.astype(jnp.int32)


def _select_indices(source_point_cloud, target_point_cloud):
    grid = (_B, _N // _SN)
    colf = jax.lax.broadcasted_iota(jnp.float32, (1, _N), 1)
    idx = pl.pallas_call(
        _select_body,
        grid=grid,
        in_specs=[
            pl.BlockSpec((1, _D, _SN), lambda b, i: (b, 0, i)),
            pl.BlockSpec((1, _D, _N), lambda b, i: (b, 0, 0)),
            pl.BlockSpec((1, _N), lambda b, i: (0, 0)),
        ],
        out_specs=pl.BlockSpec((1, 1, _SN), lambda b, i: (b, 0, i)),
        out_shape=jax.ShapeDtypeStruct((_B, 1, _N), jnp.int32),
    )(source_point_cloud, target_point_cloud, colf)
    return idx.reshape(_B, _N)


_SC_MESH = plsc.VectorSubcoreMesh(
    core_axis_name="c", subcore_axis_name="s", num_cores=_NC, num_subcores=_NS
)


@functools.partial(
    pl.kernel,
    mesh=_SC_MESH,
    compiler_params=pltpu.CompilerParams(needs_layout_passes=False),
    out_type=jax.ShapeDtypeStruct((_NW * _L,), jnp.float32),
    scratch_types=[
        [pltpu.VMEM((_N,), jnp.float32) for _ in range(_D)],
        [pltpu.VMEM((_CPW,), jnp.float32) for _ in range(_D)],
        pltpu.VMEM((_CPW,), jnp.int32),
        pltpu.VMEM((_L,), jnp.float32),
    ],
)
def _sc_rescore(src_hbm, tgt_hbm, idx_hbm, out_hbm, tgt_v, src_v, idx_v, acc_v):
    wid = lax.axis_index("s") * _NC + lax.axis_index("c")
    chunks_per_batch = _N // _CPW
    b = wid // chunks_per_batch
    chunk = wid % chunks_per_batch
    base = chunk * _CPW
    for c in range(_D):
        pltpu.sync_copy(tgt_hbm.at[pl.ds((b * _D + c) * _N, _N)], tgt_v[c])
        pltpu.sync_copy(
            src_hbm.at[pl.ds((b * _D + c) * _N + base, _CPW)], src_v[c]
        )
    pltpu.sync_copy(idx_hbm.at[pl.ds(b * _N + base, _CPW)], idx_v)

    def body(j, acc):
        iv = idx_v[pl.ds(j * _L, _L)]
        for c in range(_D):
            tc = plsc.load_gather(tgt_v[c], [iv])
            sc = src_v[c][pl.ds(j * _L, _L)]
            d = sc - tc
            acc = acc + d * d
        return acc

    acc = lax.fori_loop(0, _CPW // _L, body, jnp.zeros((_L,), jnp.float32))
    acc_v[...] = acc
    pltpu.sync_copy(acc_v, out_hbm.at[pl.ds(wid * _L, _L)])


def kernel(source_point_cloud, target_point_cloud):
    idx = _select_indices(source_point_cloud, target_point_cloud)
    partials = _sc_rescore(
        source_point_cloud.reshape(-1),
        target_point_cloud.reshape(-1),
        idx.reshape(-1),
    )
    loss = partials.reshape(_B, (_N // _CPW) * _L).sum(axis=1) * (
        1.0 / (3.0 * _N)
    )
    return loss


# speedup vs baseline: 1.7889x; 1.0299x over previous
import functools

import jax
import jax.numpy as jnp
from jax import lax
from jax.experimental import pallas as pl
from jax.experimental.pallas import tpu as pltpu
from jax.experimental.pallas import tpu_sc as plsc

_B, _D, _N = 4, 3, 4096
_SN = 512

_NC, _NS, _L = 2, 16, 16
_NW = _NC * _NS
_CPW = _B * _N // _NW


def _select_body(src_ref, tgt_ref, colf_ref, idx_ref):
    s = src_ref[0]
    t = tgt_ref[0]
    s2 = jnp.sum(s * s, axis=0)
    t2 = jnp.sum(t * t, axis=0)
    s2h = s2.astype(jnp.bfloat16)
    s2r = s2 - s2h.astype(jnp.float32)
    s2m = s2r.astype(jnp.bfloat16)
    s2l = (s2r - s2m.astype(jnp.float32)).astype(jnp.bfloat16)
    t2h = t2.astype(jnp.bfloat16)
    t2r = t2 - t2h.astype(jnp.float32)
    t2m = t2r.astype(jnp.bfloat16)
    t2l = (t2r - t2m.astype(jnp.float32)).astype(jnp.bfloat16)
    a_aug = jnp.concatenate(
        [
            s.astype(jnp.bfloat16),
            s2h[None], s2m[None], s2l[None],
            jnp.ones((3, _SN), jnp.bfloat16),
        ],
        axis=0,
    )
    b_aug = jnp.concatenate(
        [
            (t * (-2.0)).astype(jnp.bfloat16),
            jnp.ones((3, _N), jnp.bfloat16),
            t2h[None], t2m[None], t2l[None],
        ],
        axis=0,
    )
    d2_sel = jax.lax.dot_general(
        a_aug, b_aug, (((0,), (0,)), ((), ())),
        preferred_element_type=jnp.float32,
    )
    row_min = jnp.min(d2_sel, axis=1)
    colf = colf_ref[0]
    first = jnp.min(
        jnp.where(d2_sel == row_min[:, None], colf[None, :], float(_N)), axis=1
    )
    idx_ref[0, 0, :] = first.astype(jnp.int32)


def _select_indices(source_point_cloud, target_point_cloud):
    grid = (_B, _N // _SN)
    colf = jax.lax.broadcasted_iota(jnp.float32, (1, _N), 1)
    idx = pl.pallas_call(
        _select_body,
        grid=grid,
        in_specs=[
            pl.BlockSpec((1, _D, _SN), lambda b, i: (b, 0, i)),
            pl.BlockSpec((1, _D, _N), lambda b, i: (b, 0, 0)),
            pl.BlockSpec((1, _N), lambda b, i: (0, 0)),
        ],
        out_specs=pl.BlockSpec((1, 1, _SN), lambda b, i: (b, 0, i)),
        out_shape=jax.ShapeDtypeStruct((_B, 1, _N), jnp.int32),
    )(source_point_cloud, target_point_cloud, colf)
    return idx.reshape(_B, _N)


_SC_MESH = plsc.VectorSubcoreMesh(
    core_axis_name="c", subcore_axis_name="s", num_cores=_NC, num_subcores=_NS
)


@functools.partial(
    pl.kernel,
    mesh=_SC_MESH,
    compiler_params=pltpu.CompilerParams(needs_layout_passes=False),
    out_type=jax.ShapeDtypeStruct((_NW * _L,), jnp.float32),
    scratch_types=[
        [pltpu.VMEM((_N,), jnp.float32) for _ in range(_D)],
        [pltpu.VMEM((_CPW,), jnp.float32) for _ in range(_D)],
        pltpu.VMEM((_CPW,), jnp.int32),
        pltpu.VMEM((_L,), jnp.float32),
    ],
)
def _sc_rescore(src_hbm, tgt_hbm, idx_hbm, out_hbm, tgt_v, src_v, idx_v, acc_v):
    wid = lax.axis_index("s") * _NC + lax.axis_index("c")
    chunks_per_batch = _N // _CPW
    b = wid // chunks_per_batch
    chunk = wid % chunks_per_batch
    base = chunk * _CPW
    for c in range(_D):
        pltpu.sync_copy(tgt_hbm.at[pl.ds((b * _D + c) * _N, _N)], tgt_v[c])
        pltpu.sync_copy(
            src_hbm.at[pl.ds((b * _D + c) * _N + base, _CPW)], src_v[c]
        )
    pltpu.sync_copy(idx_hbm.at[pl.ds(b * _N + base, _CPW)], idx_v)

    def body(j, acc):
        iv = idx_v[pl.ds(j * _L, _L)]
        for c in range(_D):
            tc = plsc.load_gather(tgt_v[c], [iv])
            sc = src_v[c][pl.ds(j * _L, _L)]
            d = sc - tc
            acc = acc + d * d
        return acc

    acc = lax.fori_loop(0, _CPW // _L, body, jnp.zeros((_L,), jnp.float32))
    acc_v[...] = acc
    pltpu.sync_copy(acc_v, out_hbm.at[pl.ds(wid * _L, _L)])


def kernel(source_point_cloud, target_point_cloud):
    idx = _select_indices(source_point_cloud, target_point_cloud)
    partials = _sc_rescore(
        source_point_cloud.reshape(-1),
        target_point_cloud.reshape(-1),
        idx.reshape(-1),
    )
    loss = partials.reshape(_B, (_N // _CPW) * _L).sum(axis=1) * (
        1.0 / (3.0 * _N)
    )
    return loss


# speedup vs baseline: 2.4382x; 1.3630x over previous
import functools

import jax
import jax.numpy as jnp
from jax import lax
from jax.experimental import pallas as pl
from jax.experimental.pallas import tpu as pltpu
from jax.experimental.pallas import tpu_sc as plsc

_B, _D, _N = 4, 3, 4096
_SN = 1024
_NCHUNK = 32

_NC, _NS, _L = 2, 16, 16
_NW = _NC * _NS
_CPW = _B * _N // _NW


def _select_body(src_ref, tgt_ref, colf_ref, idx_ref):
    s = src_ref[0]
    t = tgt_ref[0]
    s2 = jnp.sum(s * s, axis=0)
    t2 = jnp.sum(t * t, axis=0)
    s2h = s2.astype(jnp.bfloat16)
    s2r = s2 - s2h.astype(jnp.float32)
    s2m = s2r.astype(jnp.bfloat16)
    s2l = (s2r - s2m.astype(jnp.float32)).astype(jnp.bfloat16)
    t2h = t2.astype(jnp.bfloat16)
    t2r = t2 - t2h.astype(jnp.float32)
    t2m = t2r.astype(jnp.bfloat16)
    t2l = (t2r - t2m.astype(jnp.float32)).astype(jnp.bfloat16)
    a_aug = jnp.concatenate(
        [
            s.astype(jnp.bfloat16),
            s2h[None], s2m[None], s2l[None],
            jnp.ones((3, _SN), jnp.bfloat16),
        ],
        axis=0,
    )
    b_aug = jnp.concatenate(
        [
            (t * (-2.0)).astype(jnp.bfloat16),
            jnp.ones((3, _N), jnp.bfloat16),
            t2h[None], t2m[None], t2l[None],
        ],
        axis=0,
    )
    colf = colf_ref[0]
    cw = _N // _NCHUNK
    run_v = jnp.full((_SN, cw), jnp.inf, jnp.float32)
    run_i = jnp.zeros((_SN, cw), jnp.float32)
    for k in range(_NCHUNK):
        ck = jax.lax.dot_general(
            a_aug, b_aug[:, k * cw:(k + 1) * cw], (((0,), (0,)), ((), ())),
            preferred_element_type=jnp.float32,
        )
        m = ck < run_v
        run_v = jnp.where(m, ck, run_v)
        run_i = jnp.where(m, colf[None, k * cw:(k + 1) * cw], run_i)
    row_min = jnp.min(run_v, axis=1)
    first = jnp.min(
        jnp.where(run_v == row_min[:, None], run_i, float(_N)), axis=1
    )
    idx_ref[...] = first.astype(jnp.int32)


def _select_indices(source_point_cloud, target_point_cloud):
    grid = (_B, _N // _SN)
    colf = jax.lax.broadcasted_iota(jnp.float32, (1, _N), 1)
    idx = pl.pallas_call(
        _select_body,
        grid=grid,
        in_specs=[
            pl.BlockSpec((1, _D, _SN), lambda b, i: (b, 0, i)),
            pl.BlockSpec((1, _D, _N), lambda b, i: (b, 0, 0)),
            pl.BlockSpec((1, _N), lambda b, i: (0, 0)),
        ],
        out_specs=pl.BlockSpec((_SN,), lambda b, i: (b * (_N // _SN) + i,)),
        out_shape=jax.ShapeDtypeStruct((_B * _N,), jnp.int32),
    )(source_point_cloud, target_point_cloud, colf)
    return idx


_SC_MESH = plsc.VectorSubcoreMesh(
    core_axis_name="c", subcore_axis_name="s", num_cores=_NC, num_subcores=_NS
)


@functools.partial(
    pl.kernel,
    mesh=_SC_MESH,
    compiler_params=pltpu.CompilerParams(needs_layout_passes=False),
    out_type=jax.ShapeDtypeStruct((_NW * _L,), jnp.float32),
    scratch_types=[
        [pltpu.VMEM((1, 1, _N), jnp.float32) for _ in range(_D)],
        [pltpu.VMEM((1, 1, _CPW), jnp.float32) for _ in range(_D)],
        pltpu.VMEM((_CPW,), jnp.int32),
        pltpu.VMEM((_L,), jnp.float32),
        [pltpu.SemaphoreType.DMA for _ in range(2 * _D + 1)],
    ],
)
def _sc_rescore(
    src_hbm, tgt_hbm, idx_hbm, out_hbm, tgt_v, src_v, idx_v, acc_v, sems
):
    wid = lax.axis_index("s") * _NC + lax.axis_index("c")
    chunks_per_batch = _N // _CPW
    b = wid // chunks_per_batch
    chunk = wid % chunks_per_batch
    base = chunk * _CPW
    copies = []
    for c in range(_D):
        copies.append(pltpu.async_copy(
            tgt_hbm.at[pl.ds(b, 1), pl.ds(c, 1), :], tgt_v[c], sems[2 * c]
        ))
        copies.append(pltpu.async_copy(
            src_hbm.at[pl.ds(b, 1), pl.ds(c, 1), pl.ds(base, _CPW)],
            src_v[c], sems[2 * c + 1],
        ))
    copies.append(
        pltpu.async_copy(idx_hbm.at[pl.ds(b * _N + base, _CPW)], idx_v, sems[6])
    )
    for cop in copies:
        cop.wait()
    zz = jnp.zeros((_L,), jnp.int32)
    lane = lax.iota(jnp.int32, _L)

    def body(j, acc):
        iv = idx_v[pl.ds(j * _L, _L)]
        sl = lane + j * _L
        for c in range(_D):
            tc = plsc.load_gather(tgt_v[c], [zz, zz, iv])
            sc = plsc.load_gather(src_v[c], [zz, zz, sl])
            d = sc - tc
            acc = acc + d * d
        return acc

    acc = lax.fori_loop(0, _CPW // _L, body, jnp.zeros((_L,), jnp.float32))
    acc_v[...] = acc * (1.0 / (3.0 * _N))
    pltpu.sync_copy(acc_v, out_hbm.at[pl.ds(wid * _L, _L)])


def kernel(source_point_cloud, target_point_cloud):
    idx = _select_indices(source_point_cloud, target_point_cloud)
    partials = _sc_rescore(source_point_cloud, target_point_cloud, idx)
    return partials.reshape(_B, (_N // _CPW) * _L).sum(axis=1)
